# baseline (speedup 1.0000x reference)
import functools

import jax
import jax.numpy as jnp
from jax import lax
from jax.experimental import pallas as pl
from jax.experimental.pallas import tpu as pltpu
from jax.experimental.pallas import tpu_sc as plsc

_WIDTH = 512
_SCALE = 0.001
_ROWS = 64 * 3 * 512
_NC, _NS, _L = 2, 16, 16
_NW = _NC * _NS
_RPW = _ROWS // _NW
_NBUF = 6
_CHUNK = 32
_CELEMS = _CHUNK * _WIDTH
_NCHUNK = _RPW // _CHUNK
_NGROUP = _NCHUNK // _NBUF


def _scores() -> jnp.ndarray:
    xs = jnp.linspace(-15.0, 15.0, _WIDTH)
    a = 0.0014888176096
    b = 0.0
    c = 0.0152831145355
    parabola = a * (xs - b) ** 2 + c
    parabola = parabola / parabola.sum()
    gkey = jax.random.key(42)
    u = jax.random.uniform(gkey, (_WIDTH,), minval=1e-10, maxval=1.0)
    gumbel = -jnp.log(-jnp.log(u))
    return jnp.log(parabola) + gumbel


_NEG = float(jnp.finfo(jnp.float32).min)


def _rotate(v, sh):
    idx = lax.iota(jnp.int32, _L) + sh
    idx = jnp.where(idx >= _L, idx - _L, idx)
    return v.at[idx].get(mode="promise_in_bounds")


def _all_lanes_max(v):
    for sh in (1, 2, 4, 8):
        v = jnp.maximum(v, _rotate(v, sh))
    return v


def _all_lanes_min(v):
    for sh in (1, 2, 4, 8):
        v = jnp.minimum(v, _rotate(v, sh))
    return v


def _top2(scores_v):
    nseg = _WIDTH // _L
    iota = lax.iota(jnp.int32, _L)

    def seg(j):
        return scores_v[pl.ds(j * _L, _L)], iota + j * _L

    def argmax_excluding(excl):
        m = jnp.full((_L,), _NEG, jnp.float32)
        for j in range(nseg):
            v, lanes = seg(j)
            if excl is not None:
                v = jnp.where(lanes == excl, _NEG, v)
            m = jnp.maximum(m, v)
        g = _all_lanes_max(m)
        cand = jnp.full((_L,), _WIDTH, jnp.int32)
        for j in range(nseg):
            v, lanes = seg(j)
            if excl is not None:
                v = jnp.where(lanes == excl, _NEG, v)
            cand = jnp.minimum(cand, jnp.where(v == g, lanes, _WIDTH))
        return _all_lanes_min(cand)

    i1 = argmax_excluding(None)
    i2 = argmax_excluding(i1)
    return i1, i2


def _sc_erase(scores_hbm, x_hbm, out_hbm, scores_v, bufs, sins, souts):
    wid = lax.axis_index("s") * _NC + lax.axis_index("c")
    wrow = wid * _RPW

    pltpu.sync_copy(scores_hbm, scores_v)
    i1v, i2v = _top2(scores_v)
    iota = lax.iota(jnp.int32, _L)

    def in_cp(g, b):
        return pltpu.make_async_copy(
            x_hbm.at[pl.ds(wrow + g * _CHUNK, _CHUNK), :], bufs[b], sins[b])

    def out_cp(g, b):
        return pltpu.make_async_copy(
            bufs[b], out_hbm.at[pl.ds(wrow + g * _CHUNK, _CHUNK), :], souts[b])

    def patch(b):
        buf = bufs[b]
        for j in range(_CHUNK // _L):
            rows = iota + j * _L
            for cv in (i1v, i2v):
                v = plsc.load_gather(buf, [rows, cv])
                plsc.store_scatter(buf, [rows, cv], v * jnp.float32(_SCALE))

    for b in range(_NBUF):
        in_cp(b, b).start()

    def turn(i, carry):
        g0 = i * _NBUF
        for b in range(_NBUF):
            in_cp(g0 + b, b).wait()
            patch(b)
            out_cp(g0 + b, b).start()
        for b in range(_NBUF):
            out_cp(g0 + b, b).wait()

            @pl.when(g0 + b + _NBUF < _NCHUNK)
            def _():
                in_cp(g0 + b + _NBUF, b).start()

        return carry

    lax.fori_loop(0, _NGROUP, turn, 0)


def kernel(x):
    n, c, h, w = x.shape
    x2 = x.reshape(n * c * h, w)

    def body(scores_hbm, x_hbm, out_hbm, scores_v, *rest):
        bufs = rest[:_NBUF]
        sins = rest[_NBUF:2 * _NBUF]
        souts = rest[2 * _NBUF:]
        _sc_erase(scores_hbm, x_hbm, out_hbm, scores_v, bufs, sins, souts)

    run = functools.partial(
        pl.kernel,
        out_type=jax.ShapeDtypeStruct((n * c * h, w), x.dtype),
        mesh=plsc.VectorSubcoreMesh(core_axis_name="c", subcore_axis_name="s"),
        compiler_params=pltpu.CompilerParams(
            needs_layout_passes=False, use_tc_tiling_on_sc=True),
        scratch_types=[
            pltpu.VMEM((_WIDTH,), jnp.float32),
            *[pltpu.VMEM((_CHUNK, _WIDTH), jnp.float32) for _ in range(_NBUF)],
            *[pltpu.SemaphoreType.DMA for _ in range(2 * _NBUF)],
        ],
    )(body)
    out = run(_scores(), x2)
    return out.reshape(n, c, h, w)


# speedup vs baseline: 1.0177x; 1.0177x over previous
import functools

import jax
import jax.numpy as jnp
from jax import lax
from jax.experimental import pallas as pl
from jax.experimental.pallas import tpu as pltpu
from jax.experimental.pallas import tpu_sc as plsc

_WIDTH = 512
_SCALE = 0.001
_ROWS = 64 * 3 * 512
_NC, _NS, _L = 2, 16, 16
_NW = _NC * _NS
_RPW = _ROWS // _NW
_NBUF = 4
_CHUNK = 48
_CELEMS = _CHUNK * _WIDTH
_NCHUNK = _RPW // _CHUNK
_NGROUP = _NCHUNK // _NBUF


def _scores() -> jnp.ndarray:
    xs = jnp.linspace(-15.0, 15.0, _WIDTH)
    a = 0.0014888176096
    b = 0.0
    c = 0.0152831145355
    parabola = a * (xs - b) ** 2 + c
    parabola = parabola / parabola.sum()
    gkey = jax.random.key(42)
    u = jax.random.uniform(gkey, (_WIDTH,), minval=1e-10, maxval=1.0)
    gumbel = -jnp.log(-jnp.log(u))
    return jnp.log(parabola) + gumbel


_NEG = float(jnp.finfo(jnp.float32).min)


def _rotate(v, sh):
    idx = lax.iota(jnp.int32, _L) + sh
    idx = jnp.where(idx >= _L, idx - _L, idx)
    return v.at[idx].get(mode="promise_in_bounds")


def _all_lanes_max(v):
    for sh in (1, 2, 4, 8):
        v = jnp.maximum(v, _rotate(v, sh))
    return v


def _all_lanes_min(v):
    for sh in (1, 2, 4, 8):
        v = jnp.minimum(v, _rotate(v, sh))
    return v


def _top2(scores_v):
    nseg = _WIDTH // _L
    iota = lax.iota(jnp.int32, _L)

    def seg(j):
        return scores_v[pl.ds(j * _L, _L)], iota + j * _L

    def argmax_excluding(excl):
        m = jnp.full((_L,), _NEG, jnp.float32)
        for j in range(nseg):
            v, lanes = seg(j)
            if excl is not None:
                v = jnp.where(lanes == excl, _NEG, v)
            m = jnp.maximum(m, v)
        g = _all_lanes_max(m)
        cand = jnp.full((_L,), _WIDTH, jnp.int32)
        for j in range(nseg):
            v, lanes = seg(j)
            if excl is not None:
                v = jnp.where(lanes == excl, _NEG, v)
            cand = jnp.minimum(cand, jnp.where(v == g, lanes, _WIDTH))
        return _all_lanes_min(cand)

    i1 = argmax_excluding(None)
    i2 = argmax_excluding(i1)
    return i1, i2


def _sc_erase(scores_hbm, x_hbm, out_hbm, scores_v, bufs, sins, souts):
    wid = lax.axis_index("s") * _NC + lax.axis_index("c")
    wrow = wid * _RPW

    def in_cp(g, b):
        return pltpu.make_async_copy(
            x_hbm.at[pl.ds(wrow + g * _CHUNK, _CHUNK), :], bufs[b], sins[b])

    def out_cp(g, b):
        return pltpu.make_async_copy(
            bufs[b], out_hbm.at[pl.ds(wrow + g * _CHUNK, _CHUNK), :], souts[b])

    for b in range(_NBUF):
        in_cp(b, b).start()

    pltpu.sync_copy(scores_hbm, scores_v)
    i1v, i2v = _top2(scores_v)
    iota = lax.iota(jnp.int32, _L)

    def patch(b):
        buf = bufs[b]
        for j in range(_CHUNK // _L):
            rows = iota + j * _L
            for cv in (i1v, i2v):
                v = plsc.load_gather(buf, [rows, cv])
                plsc.store_scatter(buf, [rows, cv], v * jnp.float32(_SCALE))

    def turn(i, carry):
        g0 = i * _NBUF
        for b in range(_NBUF):
            in_cp(g0 + b, b).wait()
            patch(b)
            out_cp(g0 + b, b).start()
        for b in range(_NBUF):
            out_cp(g0 + b, b).wait()

            @pl.when(g0 + b + _NBUF < _NCHUNK)
            def _():
                in_cp(g0 + b + _NBUF, b).start()

        return carry

    lax.fori_loop(0, _NGROUP, turn, 0)


def kernel(x):
    n, c, h, w = x.shape
    x2 = x.reshape(n * c * h, w)

    def body(scores_hbm, x_hbm, out_hbm, scores_v, *rest):
        bufs = rest[:_NBUF]
        sins = rest[_NBUF:2 * _NBUF]
        souts = rest[2 * _NBUF:]
        _sc_erase(scores_hbm, x_hbm, out_hbm, scores_v, bufs, sins, souts)

    run = functools.partial(
        pl.kernel,
        out_type=jax.ShapeDtypeStruct((n * c * h, w), x.dtype),
        mesh=plsc.VectorSubcoreMesh(core_axis_name="c", subcore_axis_name="s"),
        compiler_params=pltpu.CompilerParams(
            needs_layout_passes=False, use_tc_tiling_on_sc=True),
        scratch_types=[
            pltpu.VMEM((_WIDTH,), jnp.float32),
            *[pltpu.VMEM((_CHUNK, _WIDTH), jnp.float32) for _ in range(_NBUF)],
            *[pltpu.SemaphoreType.DMA for _ in range(2 * _NBUF)],
        ],
    )(body)
    out = run(_scores(), x2)
    return out.reshape(n, c, h, w)


# speedup vs baseline: 1.0190x; 1.0012x over previous
import functools

import jax
import jax.numpy as jnp
from jax import lax
from jax.experimental import pallas as pl
from jax.experimental.pallas import tpu as pltpu
from jax.experimental.pallas import tpu_sc as plsc

_WIDTH = 512
_SCALE = 0.001
_ROWS = 64 * 3 * 512
_NC, _NS, _L = 2, 16, 16
_NW = _NC * _NS
_RPW = _ROWS // _NW
_NBUF = 4
_CHUNK = 48
_NCHUNK = _RPW // _CHUNK
_NGROUP = _NCHUNK // _NBUF


def _scores() -> jnp.ndarray:
    xs = jnp.linspace(-15.0, 15.0, _WIDTH)
    a = 0.0014888176096
    b = 0.0
    c = 0.0152831145355
    parabola = a * (xs - b) ** 2 + c
    parabola = parabola / parabola.sum()
    gkey = jax.random.key(42)
    u = jax.random.uniform(gkey, (_WIDTH,), minval=1e-10, maxval=1.0)
    gumbel = -jnp.log(-jnp.log(u))
    return jnp.log(parabola) + gumbel


_NEG = float(jnp.finfo(jnp.float32).min)


def _rotate(v, sh):
    idx = lax.iota(jnp.int32, _L) + sh
    idx = jnp.where(idx >= _L, idx - _L, idx)
    return v.at[idx].get(mode="promise_in_bounds")


def _all_lanes_max(v):
    for sh in (1, 2, 4, 8):
        v = jnp.maximum(v, _rotate(v, sh))
    return v


def _all_lanes_min(v):
    for sh in (1, 2, 4, 8):
        v = jnp.minimum(v, _rotate(v, sh))
    return v


def _top2(scores_v):
    nseg = _WIDTH // _L
    iota = lax.iota(jnp.int32, _L)

    def seg(j):
        return scores_v[pl.ds(j * _L, _L)], iota + j * _L

    def argmax_excluding(excl):
        m = jnp.full((_L,), _NEG, jnp.float32)
        for j in range(nseg):
            v, lanes = seg(j)
            if excl is not None:
                v = jnp.where(lanes == excl, _NEG, v)
            m = jnp.maximum(m, v)
        g = _all_lanes_max(m)
        cand = jnp.full((_L,), _WIDTH, jnp.int32)
        for j in range(nseg):
            v, lanes = seg(j)
            if excl is not None:
                v = jnp.where(lanes == excl, _NEG, v)
            cand = jnp.minimum(cand, jnp.where(v == g, lanes, _WIDTH))
        return _all_lanes_min(cand)

    i1 = argmax_excluding(None)
    i2 = argmax_excluding(i1)
    return i1, i2


def _sc_erase(scores_hbm, x_hbm, out_hbm, scores_v, bufs, sins, souts):
    wid = lax.axis_index("s") * _NC + lax.axis_index("c")
    wrow = wid * _RPW

    def in_cp(g, b):
        return pltpu.make_async_copy(
            x_hbm.at[pl.ds(wrow + g * _CHUNK, _CHUNK), :], bufs[b], sins[b])

    def out_cp(g, b):
        return pltpu.make_async_copy(
            bufs[b], out_hbm.at[pl.ds(wrow + g * _CHUNK, _CHUNK), :], souts[b])

    for b in range(_NBUF):
        in_cp(b, b).start()

    pltpu.sync_copy(scores_hbm, scores_v)
    i1v, i2v = _top2(scores_v)
    iota = lax.iota(jnp.int32, _L)

    def patch(b):
        buf = bufs[b]
        for j in range(_CHUNK // _L):
            rows = iota + j * _L
            for cv in (i1v, i2v):
                v = plsc.load_gather(buf, [rows, cv])
                plsc.store_scatter(buf, [rows, cv], v * jnp.float32(_SCALE))

    def turn(i, carry):
        g0 = i * _NBUF
        for b in range(_NBUF):
            in_cp(g0 + b, b).wait()
            patch(b)
            out_cp(g0 + b, b).start()
        for b in range(_NBUF):
            out_cp(g0 + b, b).wait()

            @pl.when(g0 + b + _NBUF < _NCHUNK)
            def _():
                in_cp(g0 + b + _NBUF, b).start()

        return carry

    lax.fori_loop(0, _NGROUP, turn, 0)


def kernel(x):
    n, c, h, w = x.shape
    x2 = x.reshape(n * c * h, w)

    def body(scores_hbm, x_hbm, out_hbm, scores_v, *rest):
        bufs = rest[:_NBUF]
        sins = rest[_NBUF:2 * _NBUF]
        souts = rest[2 * _NBUF:]
        _sc_erase(scores_hbm, x_hbm, out_hbm, scores_v, bufs, sins, souts)

    run = functools.partial(
        pl.kernel,
        out_type=jax.ShapeDtypeStruct((n * c * h, w), x.dtype),
        mesh=plsc.VectorSubcoreMesh(core_axis_name="c", subcore_axis_name="s"),
        compiler_params=pltpu.CompilerParams(
            needs_layout_passes=False, use_tc_tiling_on_sc=True),
        scratch_types=[
            pltpu.VMEM((_WIDTH,), jnp.float32),
            *[pltpu.VMEM((_CHUNK, _WIDTH), jnp.float32) for _ in range(_NBUF)],
            *[pltpu.SemaphoreType.DMA for _ in range(2 * _NBUF)],
        ],
    )(body)
    out = run(_scores(), x2)
    return out.reshape(n, c, h, w)
